# baseline (speedup 1.0000x reference)
import functools

import jax
import jax.numpy as jnp
from jax.experimental import pallas as pl
from jax.experimental.pallas import tpu as pltpu
from jax.experimental.pallas import tpu_sc as plsc

_B = 8
_D = 32
_HW = 1024
_NUM_E = 8192
_BETA = 0.25
_CHUNK = 256
_N_CHUNKS = _NUM_E // _CHUNK
_WIN = 2048
_N_WIN = _NUM_E // _WIN
_I32_MAX = 2147483647


def _argmin_body(z_ref, zsq_ref, e_ref, esq_ref, idx_ref, row_ref):
    zb = z_ref[0].astype(jnp.bfloat16)
    z_sq = zsq_ref[0]
    row = jax.lax.broadcasted_iota(jnp.int32, (_CHUNK, _HW), 0)
    acc_v = jnp.full((1, _HW), jnp.inf, jnp.float32)
    acc_i = jnp.zeros((1, _HW), jnp.int32)
    for w in range(_N_WIN):
        best_val = jnp.full((_CHUNK, _HW), jnp.inf, jnp.float32)
        best_idx = jnp.zeros((_CHUNK, _HW), jnp.int32)
        for cc in range(_WIN // _CHUNK):
            c = w * (_WIN // _CHUNK) + cc
            e_c = e_ref[c * _CHUNK:(c + 1) * _CHUNK, :]
            e_sq = esq_ref[c * _CHUNK:(c + 1) * _CHUNK, :]
            m = jax.lax.dot_general(
                e_c, zb, (((1,), (0,)), ((), ())),
                preferred_element_type=jnp.float32)
            d = (z_sq + e_sq) - 2.0 * m
            mask = d < best_val
            best_val = jnp.where(mask, d, best_val)
            best_idx = jnp.where(mask, row + c * _CHUNK, best_idx)
        w_min = jnp.min(best_val, axis=0, keepdims=True)
        cand = jnp.where(best_val == w_min, best_idx, _I32_MAX)
        w_idx = jnp.min(cand, axis=0, keepdims=True)
        take = (w_min < acc_v) | ((w_min == acc_v) & (w_idx < acc_i))
        w_min_b = w_min.astype(jnp.bfloat16).astype(jnp.float32)
        acc_v = jnp.where(take, w_min_b, acc_v)
        acc_i = jnp.where(take, w_idx, acc_i)
    idx_ref[0] = acc_i
    row_ref[0] = acc_i >> 2


def _sc_gather(e_packed, row_flat):
    idx2 = row_flat.reshape(1, _B * _HW)
    mesh = plsc.VectorSubcoreMesh(core_axis_name="c", subcore_axis_name="s")

    @functools.partial(
        pl.kernel,
        out_type=jax.ShapeDtypeStruct((_B * _HW, 128), jnp.float32),
        mesh=mesh)
    def k(e_hbm, i_hbm, o_hbm):
        def body(i_vmem, o_vmem):
            pltpu.sync_copy(e_hbm.at[i_vmem.at[0]], o_vmem)

        pltpu.emit_pipeline(
            body,
            grid=((_B * _HW) // 128,),
            in_specs=[pl.BlockSpec((1, 128), lambda i: (0, i))],
            out_specs=[pl.BlockSpec((128, 128), lambda i: (i, 0))],
            core_axis_name=("c", "s"),
            dimension_semantics=(pltpu.PARALLEL,),
        )(i_hbm, o_hbm)

    return k(e_packed, idx2)


def _finish_body(z_ref, q_ref, idx_ref, st_ref, ps_ref):
    z = z_ref[0]
    qt_full = q_ref[...].T
    rem = idx_ref[0] & 3
    qt = qt_full[0:_D, :]
    for k in range(1, 4):
        qt = jnp.where(rem == k, qt_full[k * _D:(k + 1) * _D, :], qt)
    st_ref[0] = z + (qt - z)
    diff = z - qt
    ps_ref[...] = jnp.sum(diff * diff).reshape(1, 1, 1)


def kernel(z_e, embedding):
    z2 = z_e.reshape(_B, _D, _HW)
    z_sq = jnp.sum(z_e * z_e, axis=1).reshape(_B, 1, _HW)
    e_sq = jnp.sum(embedding * embedding, axis=1).reshape(_NUM_E, 1)
    idx3, row3 = pl.pallas_call(
        _argmin_body,
        grid=(_B,),
        in_specs=[
            pl.BlockSpec((1, _D, _HW), lambda b: (b, 0, 0)),
            pl.BlockSpec((1, 1, _HW), lambda b: (b, 0, 0)),
            pl.BlockSpec((_NUM_E, _D), lambda b: (0, 0)),
            pl.BlockSpec((_NUM_E, 1), lambda b: (0, 0)),
        ],
        out_specs=[
            pl.BlockSpec((1, 1, _HW), lambda b: (b, 0, 0)),
            pl.BlockSpec((1, 1, _HW), lambda b: (b, 0, 0)),
        ],
        out_shape=[
            jax.ShapeDtypeStruct((_B, 1, _HW), jnp.int32),
            jax.ShapeDtypeStruct((_B, 1, _HW), jnp.int32),
        ],
    )(z2, z_sq, embedding, e_sq)

    z_q_packed = _sc_gather(embedding.reshape(_NUM_E // 4, 4 * _D),
                            row3.reshape(-1))

    st, partials = pl.pallas_call(
        _finish_body,
        grid=(_B,),
        in_specs=[
            pl.BlockSpec((1, _D, _HW), lambda b: (b, 0, 0)),
            pl.BlockSpec((_HW, 4 * _D), lambda b: (b, 0)),
            pl.BlockSpec((1, 1, _HW), lambda b: (b, 0, 0)),
        ],
        out_specs=[
            pl.BlockSpec((1, _D, _HW), lambda b: (b, 0, 0)),
            pl.BlockSpec((1, 1, 1), lambda b: (b, 0, 0)),
        ],
        out_shape=[
            jax.ShapeDtypeStruct((_B, _D, _HW), jnp.float32),
            jax.ShapeDtypeStruct((_B, 1, 1), jnp.float32),
        ],
    )(z2, z_q_packed, idx3)

    mse = jnp.sum(partials) / (_B * _D * _HW)
    codebook_loss = mse
    commitment_scaled = _BETA * mse
    vq_loss = codebook_loss + commitment_scaled
    z_q_st = st.reshape(_B, _D, 32, 32)
    indices_map = idx3.reshape(_B, 32, 32)
    return (z_q_st, indices_map, vq_loss, codebook_loss, commitment_scaled)


# speedup vs baseline: 1.0216x; 1.0216x over previous
import functools

import jax
import jax.numpy as jnp
from jax.experimental import pallas as pl
from jax.experimental.pallas import tpu as pltpu
from jax.experimental.pallas import tpu_sc as plsc

_B = 8
_D = 32
_HW = 1024
_NUM_E = 8192
_BETA = 0.25
_CHUNK = 256
_N_CHUNKS = _NUM_E // _CHUNK
_WIN = 2048
_N_WIN = _NUM_E // _WIN
_I32_MAX = 2147483647


def _argmin_body(z_ref, zsq_ref, e2_ref, esq_ref, idx_ref, row_ref):
    zb = z_ref[0].astype(jnp.bfloat16)
    z_sq = zsq_ref[0]
    row = jax.lax.broadcasted_iota(jnp.int32, (_CHUNK, _HW), 0)
    acc_v = jnp.full((1, _HW), jnp.inf, jnp.float32)
    acc_i = jnp.zeros((1, _HW), jnp.int32)
    for w in range(_N_WIN):
        w_min = None
        w_idx = None
        for cc in range(_WIN // _CHUNK):
            c = w * (_WIN // _CHUNK) + cc
            e2_c = e2_ref[c * _CHUNK:(c + 1) * _CHUNK, :]
            e_sq = esq_ref[c * _CHUNK:(c + 1) * _CHUNK, :]
            m2 = jax.lax.dot_general(
                e2_c, zb, (((1,), (0,)), ((), ())),
                preferred_element_type=jnp.float32)
            d = (z_sq + e_sq) - m2
            c_min = jnp.min(d, axis=0, keepdims=True)
            cand = jnp.where(d == c_min, row + c * _CHUNK, _I32_MAX)
            c_idx = jnp.min(cand, axis=0, keepdims=True)
            if cc == 0:
                w_min, w_idx = c_min, c_idx
            else:
                t = c_min < w_min
                w_min = jnp.where(t, c_min, w_min)
                w_idx = jnp.where(t, c_idx, w_idx)
        take = (w_min < acc_v) | ((w_min == acc_v) & (w_idx < acc_i))
        w_min_b = w_min.astype(jnp.bfloat16).astype(jnp.float32)
        acc_v = jnp.where(take, w_min_b, acc_v)
        acc_i = jnp.where(take, w_idx, acc_i)
    idx_ref[0] = acc_i
    row_ref[0] = acc_i >> 2


def _sc_gather(e_packed, row_flat):
    idx2 = row_flat.reshape(1, _B * _HW)
    mesh = plsc.VectorSubcoreMesh(core_axis_name="c", subcore_axis_name="s")

    @functools.partial(
        pl.kernel,
        out_type=jax.ShapeDtypeStruct((_B * _HW, 128), jnp.float32),
        mesh=mesh)
    def k(e_hbm, i_hbm, o_hbm):
        def body(i_vmem, o_vmem):
            pltpu.sync_copy(e_hbm.at[i_vmem.at[0]], o_vmem)

        pltpu.emit_pipeline(
            body,
            grid=((_B * _HW) // 128,),
            in_specs=[pl.BlockSpec((1, 128), lambda i: (0, i))],
            out_specs=[pl.BlockSpec((128, 128), lambda i: (i, 0))],
            core_axis_name=("c", "s"),
            dimension_semantics=(pltpu.PARALLEL,),
        )(i_hbm, o_hbm)

    return k(e_packed, idx2)


def _finish_body(z_ref, q_ref, idx_ref, st_ref, ps_ref):
    z = z_ref[0]
    qt_full = q_ref[...].T
    rem = idx_ref[0] & 3
    qt = qt_full[0:_D, :]
    for k in range(1, 4):
        qt = jnp.where(rem == k, qt_full[k * _D:(k + 1) * _D, :], qt)
    st_ref[0] = z + (qt - z)
    diff = z - qt
    ps_ref[...] = jnp.sum(diff * diff).reshape(1, 1, 1)


def kernel(z_e, embedding):
    z2 = z_e.reshape(_B, _D, _HW)
    z_sq = jnp.sum(z_e * z_e, axis=1).reshape(_B, 1, _HW)
    e_sq = jnp.sum(embedding * embedding, axis=1).reshape(_NUM_E, 1)
    e2 = embedding + embedding
    idx3, row3 = pl.pallas_call(
        _argmin_body,
        grid=(_B,),
        in_specs=[
            pl.BlockSpec((1, _D, _HW), lambda b: (b, 0, 0)),
            pl.BlockSpec((1, 1, _HW), lambda b: (b, 0, 0)),
            pl.BlockSpec((_NUM_E, _D), lambda b: (0, 0)),
            pl.BlockSpec((_NUM_E, 1), lambda b: (0, 0)),
        ],
        out_specs=[
            pl.BlockSpec((1, 1, _HW), lambda b: (b, 0, 0)),
            pl.BlockSpec((1, 1, _HW), lambda b: (b, 0, 0)),
        ],
        out_shape=[
            jax.ShapeDtypeStruct((_B, 1, _HW), jnp.int32),
            jax.ShapeDtypeStruct((_B, 1, _HW), jnp.int32),
        ],
    )(z2, z_sq, e2, e_sq)

    z_q_packed = _sc_gather(embedding.reshape(_NUM_E // 4, 4 * _D),
                            row3.reshape(-1))

    st, partials = pl.pallas_call(
        _finish_body,
        grid=(_B,),
        in_specs=[
            pl.BlockSpec((1, _D, _HW), lambda b: (b, 0, 0)),
            pl.BlockSpec((_HW, 4 * _D), lambda b: (b, 0)),
            pl.BlockSpec((1, 1, _HW), lambda b: (b, 0, 0)),
        ],
        out_specs=[
            pl.BlockSpec((1, _D, _HW), lambda b: (b, 0, 0)),
            pl.BlockSpec((1, 1, 1), lambda b: (b, 0, 0)),
        ],
        out_shape=[
            jax.ShapeDtypeStruct((_B, _D, _HW), jnp.float32),
            jax.ShapeDtypeStruct((_B, 1, 1), jnp.float32),
        ],
    )(z2, z_q_packed, idx3)

    mse = jnp.sum(partials) / (_B * _D * _HW)
    codebook_loss = mse
    commitment_scaled = _BETA * mse
    vq_loss = codebook_loss + commitment_scaled
    z_q_st = st.reshape(_B, _D, 32, 32)
    indices_map = idx3.reshape(_B, 32, 32)
    return (z_q_st, indices_map, vq_loss, codebook_loss, commitment_scaled)
